# baseline (speedup 1.0000x reference)
import functools

import jax
import jax.numpy as jnp
from jax import lax
from jax.experimental import pallas as pl
from jax.experimental.pallas import tpu as pltpu
from jax.experimental.pallas import tpu_sc as plsc

SEQ = 200
BATCH = 1024
VOCAB = 100000
SPLIT = 49920
WBUF = 50080
ROWS = 104


def _make_kernel():
    nc, ns, nl = 2, 16, 16
    pb = 128
    groups = pb // nl

    mesh = plsc.VectorSubcoreMesh(core_axis_name="c", subcore_axis_name="s")

    @functools.partial(
        pl.kernel,
        mesh=mesh,
        out_type=jax.ShapeDtypeStruct((BATCH,), jnp.float32),
        compiler_params=pltpu.CompilerParams(needs_layout_passes=False),
        scratch_types=[
            pltpu.VMEM((WBUF,), jnp.float32),
            pltpu.VMEM((ROWS, pb), jnp.int32),
            pltpu.VMEM((pb,), jnp.float32),
            pltpu.VMEM((3 * pb,), jnp.float32),
            pltpu.VMEM((nl,), jnp.float32),
            pltpu.VMEM_SHARED((4 * 4 * pb,), jnp.float32),
            pltpu.SemaphoreType.DMA,
            pltpu.SemaphoreType.DMA,
        ],
    )
    def k(text_hbm, w_hbm, b_hbm, out_hbm, w_v, tok_v, part_v, peer_v,
          bias_v, shared, sem_w, sem_t):
        cid = lax.axis_index("c")
        sid = lax.axis_index("s")
        cb_local = sid // 4
        q = sid % 4
        rh = q // 2
        m = q % 2
        cb = cid * 4 + cb_local
        base = m * SPLIT

        chunk = 6272
        cps = []
        for j in range(7):
            c = lax.rem(sid + j, 7)
            cps.append(pltpu.async_copy(
                w_hbm.at[0, pl.ds(base + c * chunk, chunk)],
                w_v.at[pl.ds(c * chunk, chunk)], sem_w))
        cps.append(pltpu.async_copy(
            w_hbm.at[0, pl.ds(base + 7 * chunk, 6144)],
            w_v.at[pl.ds(7 * chunk, 6144)], sem_w))
        cps.append(pltpu.async_copy(
            w_hbm.at[0, pl.ds(base + 50048, 32)],
            w_v.at[pl.ds(50048, 32)], sem_w))
        cp_t = pltpu.async_copy(
            text_hbm.at[pl.ds(rh * 96, ROWS), pl.ds(cb * pb, pb)], tok_v,
            sem_t)
        with jax.named_scope("dma_wait"):
            cp_t.wait()
            for cp in cps:
                cp.wait()

        span = 49920 + m * 160
        lo_v = jnp.full((nl,), base, jnp.int32)
        span_u = jnp.full((nl,), span, jnp.uint32)
        max_u = jnp.full((nl,), WBUF - 1, jnp.uint32)
        zero_f = jnp.zeros((nl,), jnp.float32)

        def make_body(extra_mask):
            def body(t, accs):
                new = []
                for g in range(groups):
                    idx = tok_v[t, pl.ds(g * nl, nl)]
                    loc_u = plsc.bitcast(idx - lo_v, jnp.uint32)
                    inb = (loc_u < span_u) & extra_mask
                    loc = plsc.bitcast(jnp.minimum(loc_u, max_u), jnp.int32)
                    val = plsc.load_gather(w_v, [loc])
                    new.append(accs[g] + jnp.where(inb, val, zero_f))
                return tuple(new)
            return body

        true_v = jnp.ones((nl,), jnp.bool_)
        tail_v = jnp.full((nl,), rh == 1, jnp.bool_)
        with jax.named_scope("gather_loop"):
            accs = lax.fori_loop(0, 96, make_body(true_v),
                                 (zero_f,) * groups, unroll=2)
            accs = lax.fori_loop(96, ROWS, make_body(tail_v), accs, unroll=2)
        for g in range(groups):
            part_v[pl.ds(g * nl, nl)] = accs[g]

        slot = (cb_local * 4 + q) * pb
        pltpu.sync_copy(part_v, shared.at[pl.ds(slot, pb)])
        plsc.subcore_barrier()

        @pl.when(q == 0)
        def _():
            pltpu.sync_copy(b_hbm, bias_v.at[pl.ds(0, 1)])
            bias = bias_v[...][0]
            pltpu.sync_copy(
                shared.at[pl.ds(slot + pb, 3 * pb)], peer_v)
            for g in range(groups):
                s = pl.ds(g * nl, nl)
                part_v[s] = (part_v[s] + peer_v[s]
                             + peer_v[pl.ds(pb + g * nl, nl)]
                             + peer_v[pl.ds(2 * pb + g * nl, nl)] + bias)
            pltpu.sync_copy(part_v, out_hbm.at[pl.ds(cb * pb, pb)])

    return k


def kernel(text, W, b):
    out = _make_kernel()(text.astype(jnp.int32), W, b)
    return out.reshape(BATCH, 1)


# speedup vs baseline: 1.0135x; 1.0135x over previous
import functools

import jax
import jax.numpy as jnp
from jax import lax
from jax.experimental import pallas as pl
from jax.experimental.pallas import tpu as pltpu
from jax.experimental.pallas import tpu_sc as plsc

SEQ = 200
BATCH = 1024
VOCAB = 100000
SPLIT = 49920
WBUF = 50080
ROWS = 104


def _make_kernel():
    nc, ns, nl = 2, 16, 16
    pb = 128
    groups = pb // nl

    mesh = plsc.VectorSubcoreMesh(core_axis_name="c", subcore_axis_name="s")

    @functools.partial(
        pl.kernel,
        mesh=mesh,
        out_type=jax.ShapeDtypeStruct((BATCH,), jnp.float32),
        compiler_params=pltpu.CompilerParams(needs_layout_passes=False),
        scratch_types=[
            pltpu.VMEM((WBUF,), jnp.float32),
            pltpu.VMEM((ROWS, pb), jnp.int32),
            pltpu.VMEM((pb,), jnp.float32),
            pltpu.VMEM((3 * pb,), jnp.float32),
            pltpu.VMEM((nl,), jnp.float32),
            pltpu.VMEM_SHARED((4 * 4 * pb,), jnp.float32),
            pltpu.SemaphoreType.DMA,
            pltpu.SemaphoreType.DMA,
            pltpu.SemaphoreType.DMA,
        ],
    )
    def k(text_hbm, w_hbm, b_hbm, out_hbm, w_v, tok_v, part_v, peer_v,
          bias_v, shared, sem_w, sem_t, sem_b):
        cid = lax.axis_index("c")
        sid = lax.axis_index("s")
        cb_local = sid // 4
        q = sid % 4
        rh = q // 2
        m = q % 2
        cb = cid * 4 + cb_local
        base = m * SPLIT

        chunk = 6272
        cps = []
        for j in range(7):
            c = lax.rem(sid + j, 7)
            cps.append(pltpu.async_copy(
                w_hbm.at[0, pl.ds(base + c * chunk, chunk)],
                w_v.at[pl.ds(c * chunk, chunk)], sem_w))
        cps.append(pltpu.async_copy(
            w_hbm.at[0, pl.ds(base + 7 * chunk, 6144)],
            w_v.at[pl.ds(7 * chunk, 6144)], sem_w))
        cps.append(pltpu.async_copy(
            w_hbm.at[0, pl.ds(base + 50048, 32)],
            w_v.at[pl.ds(50048, 32)], sem_w))
        cp_t = pltpu.async_copy(
            text_hbm.at[pl.ds(rh * 96, ROWS), pl.ds(cb * pb, pb)], tok_v,
            sem_t)
        cp_b = pltpu.async_copy(b_hbm, bias_v.at[pl.ds(0, 1)], sem_b)
        with jax.named_scope("dma_wait"):
            cp_t.wait()
            for cp in cps:
                cp.wait()

        span = 49920 + m * 160
        lo_v = jnp.full((nl,), base, jnp.int32)
        span_u = jnp.full((nl,), span, jnp.uint32)
        max_u = jnp.full((nl,), WBUF - 1, jnp.uint32)
        zero_f = jnp.zeros((nl,), jnp.float32)

        def make_body(extra_mask):
            def body(t, accs):
                new = []
                for g in range(groups):
                    idx = tok_v[t, pl.ds(g * nl, nl)]
                    loc_u = plsc.bitcast(idx - lo_v, jnp.uint32)
                    inb = (loc_u < span_u) & extra_mask
                    loc = plsc.bitcast(jnp.minimum(loc_u, max_u), jnp.int32)
                    val = plsc.load_gather(w_v, [loc])
                    new.append(accs[g] + jnp.where(inb, val, zero_f))
                return tuple(new)
            return body

        true_v = jnp.ones((nl,), jnp.bool_)
        tail_v = jnp.full((nl,), rh == 1, jnp.bool_)
        with jax.named_scope("gather_loop"):
            accs = lax.fori_loop(0, 96, make_body(true_v),
                                 (zero_f,) * groups, unroll=2)
            accs = lax.fori_loop(96, ROWS, make_body(tail_v), accs, unroll=2)
        for g in range(groups):
            part_v[pl.ds(g * nl, nl)] = accs[g]

        slot = (cb_local * 4 + q) * pb
        pltpu.sync_copy(part_v, shared.at[pl.ds(slot, pb)])
        plsc.subcore_barrier()
        cp_b.wait()

        @pl.when(q == 0)
        def _():
            bias = bias_v[...][0]
            pltpu.sync_copy(
                shared.at[pl.ds(slot + pb, 3 * pb)], peer_v)
            for g in range(groups):
                s = pl.ds(g * nl, nl)
                part_v[s] = (part_v[s] + peer_v[s]
                             + peer_v[pl.ds(pb + g * nl, nl)]
                             + peer_v[pl.ds(2 * pb + g * nl, nl)] + bias)
            pltpu.sync_copy(part_v, out_hbm.at[pl.ds(cb * pb, pb)])

    return k


def kernel(text, W, b):
    out = _make_kernel()(text.astype(jnp.int32), W, b)
    return out.reshape(BATCH, 1)


# speedup vs baseline: 1.0144x; 1.0009x over previous
import functools

import jax
import jax.numpy as jnp
from jax import lax
from jax.experimental import pallas as pl
from jax.experimental.pallas import tpu as pltpu
from jax.experimental.pallas import tpu_sc as plsc

SEQ = 200
BATCH = 1024
VOCAB = 100000
SPLIT = 49920
WBUF = 50080
ROWS = 104


def _make_kernel():
    nc, ns, nl = 2, 16, 16
    pb = 128
    groups = pb // nl

    mesh = plsc.VectorSubcoreMesh(core_axis_name="c", subcore_axis_name="s")

    @functools.partial(
        pl.kernel,
        mesh=mesh,
        out_type=jax.ShapeDtypeStruct((BATCH,), jnp.float32),
        compiler_params=pltpu.CompilerParams(needs_layout_passes=False),
        scratch_types=[
            pltpu.VMEM((WBUF,), jnp.float32),
            pltpu.VMEM((ROWS, pb), jnp.int32),
            pltpu.VMEM((pb,), jnp.float32),
            pltpu.VMEM((3 * pb,), jnp.float32),
            pltpu.VMEM((nl,), jnp.float32),
            pltpu.VMEM_SHARED((4 * 4 * pb,), jnp.float32),
            pltpu.SemaphoreType.DMA,
            pltpu.SemaphoreType.DMA,
            pltpu.SemaphoreType.DMA,
        ],
    )
    def k(text_hbm, w_hbm, b_hbm, out_hbm, w_v, tok_v, part_v, peer_v,
          bias_v, shared, sem_w, sem_t, sem_b):
        cid = lax.axis_index("c")
        sid = lax.axis_index("s")
        cb_local = sid // 4
        q = sid % 4
        rh = q // 2
        m = q % 2
        cb = cid * 4 + cb_local
        base = m * SPLIT

        chunk = 6272
        cps = []
        for j in range(7):
            c = lax.rem(sid + j, 7)
            cps.append(pltpu.async_copy(
                w_hbm.at[0, pl.ds(base + c * chunk, chunk)],
                w_v.at[pl.ds(c * chunk, chunk)], sem_w))
        cps.append(pltpu.async_copy(
            w_hbm.at[0, pl.ds(base + 7 * chunk, 6144)],
            w_v.at[pl.ds(7 * chunk, 6144)], sem_w))
        cps.append(pltpu.async_copy(
            w_hbm.at[0, pl.ds(base + 50048, 32)],
            w_v.at[pl.ds(50048, 32)], sem_w))
        cp_t = pltpu.async_copy(
            text_hbm.at[pl.ds(rh * 96, ROWS), pl.ds(cb * pb, pb)], tok_v,
            sem_t)
        cp_b = pltpu.async_copy(b_hbm, bias_v.at[pl.ds(0, 1)], sem_b)
        with jax.named_scope("dma_wait"):
            cp_t.wait()
            for cp in cps:
                cp.wait()

        span = 49920 + m * 160
        lo_v = jnp.full((nl,), base, jnp.int32)
        span_u = jnp.full((nl,), span, jnp.uint32)
        max_u = jnp.full((nl,), WBUF - 1, jnp.uint32)
        zero_f = jnp.zeros((nl,), jnp.float32)

        def make_body(extra_mask):
            def body(t, accs):
                new = []
                for g in range(groups):
                    idx = tok_v[t, pl.ds(g * nl, nl)]
                    loc_u = plsc.bitcast(idx - lo_v, jnp.uint32)
                    inb = (loc_u < span_u) & extra_mask
                    loc = plsc.bitcast(jnp.minimum(loc_u, max_u), jnp.int32)
                    val = plsc.load_gather(w_v, [loc])
                    new.append(accs[g] + jnp.where(inb, val, zero_f))
                return tuple(new)
            return body

        true_v = jnp.ones((nl,), jnp.bool_)
        tail_v = jnp.full((nl,), rh == 1, jnp.bool_)
        with jax.named_scope("gather_loop"):
            accs = lax.fori_loop(0, 96, make_body(true_v),
                                 (zero_f,) * groups, unroll=1)
            accs = lax.fori_loop(96, ROWS, make_body(tail_v), accs, unroll=1)
        for g in range(groups):
            part_v[pl.ds(g * nl, nl)] = accs[g]

        slot = (cb_local * 4 + q) * pb
        pltpu.sync_copy(part_v, shared.at[pl.ds(slot, pb)])
        plsc.subcore_barrier()
        cp_b.wait()

        @pl.when(q == 0)
        def _():
            bias = bias_v[...][0]
            pltpu.sync_copy(
                shared.at[pl.ds(slot + pb, 3 * pb)], peer_v)
            for g in range(groups):
                s = pl.ds(g * nl, nl)
                part_v[s] = (part_v[s] + peer_v[s]
                             + peer_v[pl.ds(pb + g * nl, nl)]
                             + peer_v[pl.ds(2 * pb + g * nl, nl)] + bias)
            pltpu.sync_copy(part_v, out_hbm.at[pl.ds(cb * pb, pb)])

    return k


def kernel(text, W, b):
    out = _make_kernel()(text.astype(jnp.int32), W, b)
    return out.reshape(BATCH, 1)


# speedup vs baseline: 1.0177x; 1.0032x over previous
import functools

import jax
import jax.numpy as jnp
from jax import lax
from jax.experimental import pallas as pl
from jax.experimental.pallas import tpu as pltpu
from jax.experimental.pallas import tpu_sc as plsc

SEQ = 200
BATCH = 1024
VOCAB = 100000
SPLIT = 49920
WBUF = 50080
ROWS = 104


def _make_kernel():
    nc, ns, nl = 2, 16, 16
    pb = 128
    groups = pb // nl

    mesh = plsc.VectorSubcoreMesh(core_axis_name="c", subcore_axis_name="s")

    @functools.partial(
        pl.kernel,
        mesh=mesh,
        out_type=jax.ShapeDtypeStruct((BATCH,), jnp.float32),
        compiler_params=pltpu.CompilerParams(
            needs_layout_passes=False, skip_device_barrier=True
        ),
        scratch_types=[
            pltpu.VMEM((WBUF,), jnp.float32),
            pltpu.VMEM((ROWS, pb), jnp.int32),
            pltpu.VMEM((pb,), jnp.float32),
            pltpu.VMEM((3 * pb,), jnp.float32),
            pltpu.VMEM((nl,), jnp.float32),
            pltpu.VMEM_SHARED((4 * 4 * pb,), jnp.float32),
            pltpu.SemaphoreType.DMA,
            pltpu.SemaphoreType.DMA,
            pltpu.SemaphoreType.DMA,
        ],
    )
    def k(text_hbm, w_hbm, b_hbm, out_hbm, w_v, tok_v, part_v, peer_v,
          bias_v, shared, sem_w, sem_t, sem_b):
        cid = lax.axis_index("c")
        sid = lax.axis_index("s")
        cb_local = sid // 4
        q = sid % 4
        rh = q // 2
        m = q % 2
        cb = cid * 4 + cb_local
        base = m * SPLIT

        chunk = 6272
        cps = []
        for j in range(7):
            c = lax.rem(sid + j, 7)
            cps.append(pltpu.async_copy(
                w_hbm.at[0, pl.ds(base + c * chunk, chunk)],
                w_v.at[pl.ds(c * chunk, chunk)], sem_w))
        cps.append(pltpu.async_copy(
            w_hbm.at[0, pl.ds(base + 7 * chunk, 6144)],
            w_v.at[pl.ds(7 * chunk, 6144)], sem_w))
        cps.append(pltpu.async_copy(
            w_hbm.at[0, pl.ds(base + 50048, 32)],
            w_v.at[pl.ds(50048, 32)], sem_w))
        cp_t = pltpu.async_copy(
            text_hbm.at[pl.ds(rh * 96, ROWS), pl.ds(cb * pb, pb)], tok_v,
            sem_t)
        cp_b = pltpu.async_copy(b_hbm, bias_v.at[pl.ds(0, 1)], sem_b)
        with jax.named_scope("dma_wait"):
            cp_t.wait()
            for cp in cps:
                cp.wait()

        span = 49920 + m * 160
        lo_v = jnp.full((nl,), base, jnp.int32)
        span_u = jnp.full((nl,), span, jnp.uint32)
        max_u = jnp.full((nl,), WBUF - 1, jnp.uint32)
        zero_f = jnp.zeros((nl,), jnp.float32)

        def make_body(extra_mask):
            def body(t, accs):
                new = []
                for g in range(groups):
                    idx = tok_v[t, pl.ds(g * nl, nl)]
                    loc_u = plsc.bitcast(idx - lo_v, jnp.uint32)
                    inb = (loc_u < span_u) & extra_mask
                    loc = plsc.bitcast(jnp.minimum(loc_u, max_u), jnp.int32)
                    val = plsc.load_gather(w_v, [loc])
                    new.append(accs[g] + jnp.where(inb, val, zero_f))
                return tuple(new)
            return body

        true_v = jnp.ones((nl,), jnp.bool_)
        tail_v = jnp.full((nl,), rh == 1, jnp.bool_)
        with jax.named_scope("gather_loop"):
            accs = lax.fori_loop(0, 96, make_body(true_v),
                                 (zero_f,) * groups, unroll=1)
            accs = lax.fori_loop(96, ROWS, make_body(tail_v), accs, unroll=1)
        for g in range(groups):
            part_v[pl.ds(g * nl, nl)] = accs[g]

        slot = (cb_local * 4 + q) * pb
        pltpu.sync_copy(part_v, shared.at[pl.ds(slot, pb)])
        plsc.subcore_barrier()
        cp_b.wait()

        @pl.when(q == 0)
        def _():
            bias = bias_v[...][0]
            pltpu.sync_copy(
                shared.at[pl.ds(slot + pb, 3 * pb)], peer_v)
            for g in range(groups):
                s = pl.ds(g * nl, nl)
                part_v[s] = (part_v[s] + peer_v[s]
                             + peer_v[pl.ds(pb + g * nl, nl)]
                             + peer_v[pl.ds(2 * pb + g * nl, nl)] + bias)
            pltpu.sync_copy(part_v, out_hbm.at[pl.ds(cb * pb, pb)])

    return k


def kernel(text, W, b):
    out = _make_kernel()(text.astype(jnp.int32), W, b)
    return out.reshape(BATCH, 1)


# speedup vs baseline: 1.0178x; 1.0000x over previous
import functools

import jax
import jax.numpy as jnp
from jax import lax
from jax.experimental import pallas as pl
from jax.experimental.pallas import tpu as pltpu
from jax.experimental.pallas import tpu_sc as plsc

SEQ = 200
BATCH = 1024
VOCAB = 100000
SPLIT = 49920
WBUF = 50080
ROWS = 104


def _make_kernel():
    nc, ns, nl = 2, 16, 16
    pb = 128
    groups = pb // nl

    mesh = plsc.VectorSubcoreMesh(core_axis_name="c", subcore_axis_name="s")

    @functools.partial(
        pl.kernel,
        mesh=mesh,
        out_type=jax.ShapeDtypeStruct((BATCH,), jnp.float32),
        compiler_params=pltpu.CompilerParams(
            needs_layout_passes=False, skip_device_barrier=True
        ),
        scratch_types=[
            pltpu.VMEM((WBUF,), jnp.float32),
            pltpu.VMEM((ROWS, pb), jnp.int32),
            pltpu.VMEM((pb,), jnp.float32),
            pltpu.VMEM((3 * pb,), jnp.float32),
            pltpu.VMEM((nl,), jnp.float32),
            pltpu.VMEM_SHARED((4 * 4 * pb,), jnp.float32),
            pltpu.SemaphoreType.DMA,
            pltpu.SemaphoreType.DMA,
            pltpu.SemaphoreType.DMA,
        ],
    )
    def k(text_hbm, w_hbm, b_hbm, out_hbm, w_v, tok_v, part_v, peer_v,
          bias_v, shared, sem_w, sem_t, sem_b):
        cid = lax.axis_index("c")
        sid = lax.axis_index("s")
        cb_local = sid // 4
        q = sid % 4
        rh = q // 2
        m = q % 2
        cb = cid * 4 + cb_local
        base = m * SPLIT

        chunk = 6272
        cps = []
        for j in range(7):
            c = lax.rem(sid + j, 7)
            cps.append(pltpu.async_copy(
                w_hbm.at[0, pl.ds(base + c * chunk, chunk)],
                w_v.at[pl.ds(c * chunk, chunk)], sem_w))
        cps.append(pltpu.async_copy(
            w_hbm.at[0, pl.ds(base + 7 * chunk, 6144)],
            w_v.at[pl.ds(7 * chunk, 6144)], sem_w))
        cps.append(pltpu.async_copy(
            w_hbm.at[0, pl.ds(base + 50048, 32)],
            w_v.at[pl.ds(50048, 32)], sem_w))
        cp_t = pltpu.async_copy(
            text_hbm.at[pl.ds(rh * 96, ROWS), pl.ds(cb * pb, pb)], tok_v,
            sem_t)
        cp_b = pltpu.async_copy(b_hbm, bias_v.at[pl.ds(0, 1)], sem_b)
        cp_t.wait()
        for cp in cps:
            cp.wait()

        span = 49920 + m * 160
        lo_v = jnp.full((nl,), base, jnp.int32)
        span_u = jnp.full((nl,), span, jnp.uint32)
        max_u = jnp.full((nl,), WBUF - 1, jnp.uint32)
        zero_f = jnp.zeros((nl,), jnp.float32)

        def make_body(extra_mask):
            def body(t, accs):
                new = []
                for g in range(groups):
                    idx = tok_v[t, pl.ds(g * nl, nl)]
                    loc_u = plsc.bitcast(idx - lo_v, jnp.uint32)
                    inb = (loc_u < span_u) & extra_mask
                    loc = plsc.bitcast(jnp.minimum(loc_u, max_u), jnp.int32)
                    val = plsc.load_gather(w_v, [loc])
                    new.append(accs[g] + jnp.where(inb, val, zero_f))
                return tuple(new)
            return body

        true_v = jnp.ones((nl,), jnp.bool_)
        tail_v = jnp.full((nl,), rh == 1, jnp.bool_)
        accs = lax.fori_loop(0, 96, make_body(true_v),
                             (zero_f,) * groups, unroll=1)
        accs = lax.fori_loop(96, ROWS, make_body(tail_v), accs, unroll=1)
        for g in range(groups):
            part_v[pl.ds(g * nl, nl)] = accs[g]

        slot = (cb_local * 4 + q) * pb
        pltpu.sync_copy(part_v, shared.at[pl.ds(slot, pb)])
        plsc.subcore_barrier()
        cp_b.wait()

        @pl.when(q == 0)
        def _():
            bias = bias_v[...][0]
            pltpu.sync_copy(
                shared.at[pl.ds(slot + pb, 3 * pb)], peer_v)
            for g in range(groups):
                s = pl.ds(g * nl, nl)
                part_v[s] = (part_v[s] + peer_v[s]
                             + peer_v[pl.ds(pb + g * nl, nl)]
                             + peer_v[pl.ds(2 * pb + g * nl, nl)] + bias)
            pltpu.sync_copy(part_v, out_hbm.at[pl.ds(cb * pb, pb)])

    return k


def kernel(text, W, b):
    out = _make_kernel()(text.astype(jnp.int32), W, b)
    return out.reshape(BATCH, 1)
